# baseline (speedup 1.0000x reference)
import functools

import jax
import jax.numpy as jnp
from jax import lax
from jax.experimental import pallas as pl
from jax.experimental.pallas import tpu as pltpu
from jax.experimental.pallas import tpu_sc as plsc

_NC = 2
_NS = 16
_NW = _NC * _NS
_LANES = 16
_TV = 1024


_ROWS_PER_STEP = 16


def _prep_body(v_ref, u_ref, cen_ref, cnt_ref, g_ref, idx_ref):
    voc = u_ref.shape[0]
    rows = lax.dot_general(
        v_ref[...], u_ref[...], (((1,), (1,)), ((), ())),
        preferred_element_type=jnp.float32,
        precision=lax.Precision.HIGHEST)
    padded = jnp.pad(rows, ((0, 0), (0, _TV - voc)))
    g_ref[...] = padded.reshape(_ROWS_PER_STEP * _TV)

    @pl.when(pl.program_id(0) == 0)
    def _():
        idx_ref[...] = cen_ref[...] * _TV + cnt_ref[...]


@functools.lru_cache(maxsize=None)
def _make_sc_gather(B, K1):
    bpw = (B * K1) // _NW
    n_chunk = bpw // 128
    n_red = bpw // _LANES
    mesh = plsc.VectorSubcoreMesh(core_axis_name="c", subcore_axis_name="s")

    @functools.partial(
        pl.kernel, mesh=mesh,
        out_type=jax.ShapeDtypeStruct((_NW, 2 * _LANES), jnp.float32),
        scratch_types=[
            pltpu.VMEM((bpw,), jnp.int32),
            pltpu.VMEM((bpw,), jnp.float32),
            pltpu.VMEM((2 * _LANES,), jnp.float32),
            pltpu.SemaphoreType.DMA,
        ])
    def sc_fn(g_hbm, idx_hbm, out_hbm, idx_v, val_v, st_v, sem):
        wid = lax.axis_index("s") * _NC + lax.axis_index("c")
        base = wid * bpw
        pltpu.sync_copy(idx_hbm.at[pl.ds(base, bpw)], idx_v)

        copies = [
            pltpu.async_copy(
                g_hbm.at[idx_v.at[pl.ds(j * 128, 128)]],
                val_v.at[pl.ds(j * 128, 128)], sem)
            for j in range(n_chunk)
        ]
        for cp in copies:
            cp.wait()

        zero = jnp.zeros((_LANES,), jnp.float32)
        acc = zero
        acc0 = zero
        for i in range(n_red):
            x = val_v[pl.ds(i * _LANES, _LANES)]
            acc = acc + jnp.exp(x)
            acc0 = acc0 + jnp.where(base + i * _LANES < B, x, zero)
        st_v[pl.ds(0, _LANES)] = acc
        st_v[pl.ds(_LANES, _LANES)] = acc0
        pltpu.sync_copy(st_v, out_hbm.at[wid])

    return sc_fn


@functools.lru_cache(maxsize=None)
def _make_finalize(B):
    def _fin_body(p_ref, out_ref):
        s = jnp.sum(p_ref[:, 0:_LANES])
        l0 = jnp.sum(p_ref[:, _LANES:2 * _LANES])
        out_ref[...] = jnp.reshape(float(B) * jnp.log(s) - l0, (1, 1))

    return pl.pallas_call(
        _fin_body,
        out_shape=jax.ShapeDtypeStruct((1, 1), jnp.float32))


def kernel(V, U, centers, contexts_negs):
    voc, d = V.shape
    B = centers.shape[0]
    K1 = contexts_negs.shape[1]
    n_steps = (voc + _ROWS_PER_STEP - 1) // _ROWS_PER_STEP
    G_flat, idx = pl.pallas_call(
        _prep_body,
        grid=(n_steps,),
        in_specs=[
            pl.BlockSpec((_ROWS_PER_STEP, d), lambda i: (i, 0)),
            pl.BlockSpec((voc, d), lambda i: (0, 0)),
            pl.BlockSpec((1, B), lambda i: (0, 0)),
            pl.BlockSpec((K1, B), lambda i: (0, 0)),
        ],
        out_specs=(pl.BlockSpec((_ROWS_PER_STEP * _TV,), lambda i: (i,)),
                   pl.BlockSpec((K1, B), lambda i: (0, 0))),
        out_shape=(jax.ShapeDtypeStruct((_TV * _TV,), jnp.float32),
                   jax.ShapeDtypeStruct((K1, B), jnp.int32)),
    )(V, U, centers[None, :], contexts_negs.T)
    esum_lsum = _make_sc_gather(B, K1)(G_flat, idx.reshape(K1 * B))
    loss = _make_finalize(B)(esum_lsum)
    return loss[0, 0]


# speedup vs baseline: 1.9135x; 1.9135x over previous
import functools

import jax
import jax.numpy as jnp
from jax import lax
from jax.experimental import pallas as pl
from jax.experimental.pallas import tpu as pltpu
from jax.experimental.pallas import tpu_sc as plsc

_NC = 2
_NS = 16
_NW = _NC * _NS
_LANES = 16
_TV = 1024


def _prep_body(v_ref, u_ref, cen_ref, cnt_ref, g_ref, idx_ref):
    voc = v_ref.shape[0]
    g_ref[0:voc, 0:voc] = lax.dot_general(
        v_ref[...], u_ref[...], (((1,), (1,)), ((), ())),
        preferred_element_type=jnp.float32,
        precision=lax.Precision.HIGHEST)
    idx_ref[...] = cen_ref[...] * _TV + cnt_ref[...]


@functools.lru_cache(maxsize=None)
def _make_sc_gather(B, K1):
    bcols = B // _NW
    bpw = bcols * K1
    cpk = bcols // 128
    rpc = 128 // _LANES
    mesh = plsc.VectorSubcoreMesh(core_axis_name="c", subcore_axis_name="s")

    @functools.partial(
        pl.kernel, mesh=mesh,
        out_type=jax.ShapeDtypeStruct((_NW, 2 * _LANES), jnp.float32),
        scratch_types=[
            pltpu.VMEM((K1, bcols), jnp.int32),
            pltpu.VMEM((bpw,), jnp.float32),
            pltpu.VMEM((2 * _LANES,), jnp.float32),
            pltpu.SemaphoreType.DMA,
            pltpu.SemaphoreType.DMA,
        ])
    def sc_fn(g_hbm, idx_hbm, out_hbm, idx_v, val_v, st_v, isem, gsem):
        wid = lax.axis_index("s") * _NC + lax.axis_index("c")
        base = wid * bcols
        idx_cps = [
            pltpu.async_copy(idx_hbm.at[k, pl.ds(base, bcols)],
                             idx_v.at[k], isem)
            for k in range(K1)
        ]
        for cp in idx_cps:
            cp.wait()

        copies = [
            pltpu.async_copy(
                g_hbm.at[idx_v.at[k, pl.ds(c * 128, 128)]],
                val_v.at[pl.ds((k * cpk + c) * 128, 128)], gsem)
            for k in range(K1) for c in range(cpk)
        ]

        zero = jnp.zeros((_LANES,), jnp.float32)
        acc = zero
        acc0 = zero
        for j, cp in enumerate(copies):
            cp.wait()
            for t in range(rpc):
                x = val_v[pl.ds(j * 128 + t * _LANES, _LANES)]
                acc = acc + jnp.exp(x)
                if j < cpk:
                    acc0 = acc0 + x
        st_v[pl.ds(0, _LANES)] = acc
        st_v[pl.ds(_LANES, _LANES)] = acc0
        pltpu.sync_copy(st_v, out_hbm.at[wid])

    return sc_fn


@functools.lru_cache(maxsize=None)
def _make_finalize(B):
    def _fin_body(p_ref, out_ref):
        s = jnp.sum(p_ref[:, 0:_LANES])
        l0 = jnp.sum(p_ref[:, _LANES:2 * _LANES])
        out_ref[...] = jnp.reshape(float(B) * jnp.log(s) - l0, (1, 1))

    return pl.pallas_call(
        _fin_body,
        out_shape=jax.ShapeDtypeStruct((1, 1), jnp.float32))


def kernel(V, U, centers, contexts_negs):
    voc, d = V.shape
    B = centers.shape[0]
    K1 = contexts_negs.shape[1]
    G, idx = pl.pallas_call(
        _prep_body,
        out_shape=(jax.ShapeDtypeStruct((_TV, _TV), jnp.float32),
                   jax.ShapeDtypeStruct((K1, B), jnp.int32)),
    )(V, U, centers[None, :], contexts_negs.T)
    esum_lsum = _make_sc_gather(B, K1)(G.reshape(_TV * _TV), idx)
    loss = _make_finalize(B)(esum_lsum)
    return loss[0, 0]


# speedup vs baseline: 1.9415x; 1.0146x over previous
import functools

import jax
import jax.numpy as jnp
from jax import lax
from jax.experimental import pallas as pl
from jax.experimental.pallas import tpu as pltpu
from jax.experimental.pallas import tpu_sc as plsc

_NC = 2
_NS = 16
_NW = _NC * _NS
_LANES = 16
_TV = 1024


def _prep_body(v_ref, u_ref, cen_ref, cnt_ref, g_ref, idx_ref):
    voc = v_ref.shape[0]
    g_ref[0:voc, 0:voc] = lax.dot_general(
        v_ref[...], u_ref[...], (((1,), (1,)), ((), ())),
        preferred_element_type=jnp.float32,
        precision=lax.Precision.HIGHEST)
    idx_ref[...] = cen_ref[...] * _TV + cnt_ref[...]


@functools.lru_cache(maxsize=None)
def _make_sc_gather(B, K1):
    bcols = B // _NW
    bpw = bcols * K1
    cpk = bcols // 128
    rpc = 128 // _LANES
    mesh = plsc.VectorSubcoreMesh(core_axis_name="c", subcore_axis_name="s")

    @functools.partial(
        pl.kernel, mesh=mesh,
        out_type=jax.ShapeDtypeStruct((_NW, 2 * _LANES), jnp.float32),
        scratch_types=[
            pltpu.VMEM((bpw,), jnp.int32),
            pltpu.VMEM((bpw,), jnp.float32),
            pltpu.VMEM((2 * _LANES,), jnp.float32),
            pltpu.SemaphoreType.DMA,
            pltpu.SemaphoreType.DMA,
        ])
    def sc_fn(g_hbm, idx_hbm, out_hbm, idx_v, val_v, st_v, isem, gsem):
        wid = lax.axis_index("s") * _NC + lax.axis_index("c")
        base = wid * bcols
        idx_cps = [
            pltpu.async_copy(idx_hbm.at[k, pl.ds(base, bcols)],
                             idx_v.at[pl.ds(k * bcols, bcols)], isem)
            for k in range(K1)
        ]
        for cp in idx_cps:
            cp.wait()

        pltpu.async_copy(g_hbm.at[idx_v], val_v, gsem).wait()

        zero = jnp.zeros((_LANES,), jnp.float32)
        acc = zero
        acc0 = zero
        for i in range(bpw // _LANES):
            x = val_v[pl.ds(i * _LANES, _LANES)]
            acc = acc + jnp.exp(x)
            if i < bcols // _LANES:
                acc0 = acc0 + x
        st_v[pl.ds(0, _LANES)] = acc
        st_v[pl.ds(_LANES, _LANES)] = acc0
        pltpu.sync_copy(st_v, out_hbm.at[wid])

    return sc_fn


@functools.lru_cache(maxsize=None)
def _make_finalize(B):
    def _fin_body(p_ref, out_ref):
        s = jnp.sum(p_ref[:, 0:_LANES])
        l0 = jnp.sum(p_ref[:, _LANES:2 * _LANES])
        out_ref[...] = jnp.reshape(float(B) * jnp.log(s) - l0, (1, 1))

    return pl.pallas_call(
        _fin_body,
        out_shape=jax.ShapeDtypeStruct((1, 1), jnp.float32))


def kernel(V, U, centers, contexts_negs):
    voc, d = V.shape
    B = centers.shape[0]
    K1 = contexts_negs.shape[1]
    G, idx = pl.pallas_call(
        _prep_body,
        out_shape=(jax.ShapeDtypeStruct((_TV, _TV), jnp.float32),
                   jax.ShapeDtypeStruct((K1, B), jnp.int32)),
    )(V, U, centers[None, :], contexts_negs.T)
    esum_lsum = _make_sc_gather(B, K1)(G.reshape(_TV * _TV), idx)
    loss = _make_finalize(B)(esum_lsum)
    return loss[0, 0]
